# baseline (speedup 1.0000x reference)
import functools

import jax
import jax.numpy as jnp
from jax import lax
from jax.experimental import pallas as pl
from jax.experimental.pallas import tpu as pltpu
from jax.experimental.pallas import tpu_sc as plsc

_N = 10000
_DIN = 128
_DP = 3
_K = 16
_DOUT = 128
_NPAD = 10240
_PT = _NPAD // 32
_PB = 8
_SLOW_CORE = 0
_NSB_S = 24
_NSB_F = 56
_NG = _DP + 1
_DG = _NG * _DIN
_NVR = _DIN // 16
_BN = 1024


def _rows_body(f_ref, ft_ref):
    ft_ref[...] = f_ref[...].T


def _build_rows(f_pad):
    return pl.pallas_call(
        _rows_body,
        grid=(_NPAD // _BN,),
        in_specs=[pl.BlockSpec((_DIN, _BN), lambda i: (0, i))],
        out_specs=pl.BlockSpec((_BN, _DIN), lambda i: (i, 0)),
        out_shape=jax.ShapeDtypeStruct((_NPAD, _DIN), jnp.float32),
    )(f_pad)


def _sc_body(ft_hbm, w4_hbm, nbr_hbm, out_hbm, idx_v, fbufs, w4_v, obufs,
             gsems, osems):
    cid = lax.axis_index("c")
    sid = lax.axis_index("s")
    rows = _PB * _K
    is_slow = cid == _SLOW_CORE
    nsb = jnp.where(is_slow, _NSB_S, _NSB_F)
    base = jnp.where(is_slow, sid * (_NSB_S * _PB),
                     _NSB_S * _PB * 16 + sid * (_NSB_F * _PB))

    with jax.named_scope("sc_init"):
        pltpu.sync_copy(w4_hbm, w4_v)
        pltpu.sync_copy(nbr_hbm.at[pl.ds(base * _K, _NSB_F * _PB * _K)],
                        idx_v)

    def start_gather(s, b):
        pltpu.async_copy(ft_hbm.at[idx_v.at[pl.ds(s * rows, rows)]],
                         fbufs.at[b], gsems[b])

    def compute(s, b):
        with jax.named_scope("sc_compute"):
            _compute_inner(s, b)

    def _compute_inner(s, b):
        @pl.loop(0, _PB)
        def _point(p):
            r0 = p * _K
            idxp = idx_v[pl.ds(s * rows + r0, _K)]
            wvec = [plsc.load_gather(w4_v, [idxp + g * _NPAD])
                    for g in range(_DP)]
            acc = [[jnp.zeros((16,), jnp.float32) for _ in range(_NVR)]
                   for _ in range(_NG)]
            for k in range(_K):
                fv = [fbufs[b, r0 + k, pl.ds(j * 16, 16)]
                      for j in range(_NVR)]
                for j in range(_NVR):
                    acc[0][j] = acc[0][j] + fv[j]
                for g in range(_DP):
                    wgk = wvec[g][k]
                    for j in range(_NVR):
                        acc[g + 1][j] = acc[g + 1][j] + wgk * fv[j]
            for g in range(_NG):
                for j in range(_NVR):
                    obufs[b, p, pl.ds(g * _DIN + j * 16, 16)] = acc[g][j]

    def wait_gather(b):
        with jax.named_scope("sc_wait_gather"):
            pltpu.make_async_copy(ft_hbm.at[idx_v.at[pl.ds(0, rows)]],
                                  fbufs.at[b], gsems[b]).wait()

    def start_store(s, b):
        pltpu.async_copy(obufs.at[b], out_hbm.at[pl.ds(base + s * _PB, _PB)],
                         osems[b])

    def wait_store(s, b):
        pltpu.make_async_copy(obufs.at[b],
                              out_hbm.at[pl.ds(base + s * _PB, _PB)],
                              osems[b]).wait()

    nbuf = 4
    with jax.named_scope("sc_prime"):
        for b in range(nbuf - 1):
            start_gather(b, b)

    @pl.loop(0, nsb, step=nbuf)
    def _quad(s):
        for b in range(nbuf):
            pf = s + b + nbuf - 1

            @pl.when(pf < nsb)
            def _():
                start_gather(pf, (b + nbuf - 1) % nbuf)

            wait_gather(b)

            @pl.when(s >= nbuf)
            def _():
                wait_store(s + b - nbuf, b)

            compute(s + b, b)
            start_store(s + b, b)

    for b in range(nbuf):
        wait_store(nsb - nbuf + b, b)


def _sc_gather(ft, w4, nbr_pm):
    mesh = plsc.VectorSubcoreMesh(core_axis_name="c", subcore_axis_name="s")
    return pl.kernel(
        _sc_body,
        out_type=jax.ShapeDtypeStruct((_NPAD, _DG), jnp.float32),
        mesh=mesh,
        compiler_params=pltpu.CompilerParams(needs_layout_passes=False),
        scratch_types=[
            pltpu.VMEM((_NSB_F * _PB * _K,), jnp.int32),
            pltpu.VMEM((4, _PB * _K, _DIN), jnp.float32),
            pltpu.VMEM((_DP * _NPAD,), jnp.float32),
            pltpu.VMEM((4, _PB, _DG), jnp.float32),
            [pltpu.SemaphoreType.DMA] * 4,
            [pltpu.SemaphoreType.DMA] * 4,
        ],
    )(ft, w4, nbr_pm)


def _y_body(a_ref, pt_ref, w_ref, fb_ref, y_ref):
    a = a_ref[...]
    pt = pt_ref[...]
    a0 = a[:, 0:_DIN]
    parts = [a0]
    for p in range(_DP):
        parts.append(a[:, (p + 1) * _DIN:(p + 2) * _DIN] - a0 * pt[:, p:p + 1])
    ahat = jnp.concatenate(parts, axis=1)
    w = w_ref[...]
    y_t = lax.dot_general(w, ahat, (((0,), (1,)), ((), ())),
                          preferred_element_type=jnp.float32)
    y_ref[...] = y_t + fb_ref[...]


def _contract(a, pt_pad, w, fb):
    return pl.pallas_call(
        _y_body,
        grid=(_NPAD // _BN,),
        in_specs=[
            pl.BlockSpec((_BN, _DG), lambda i: (i, 0)),
            pl.BlockSpec((_BN, 8), lambda i: (i, 0)),
            pl.BlockSpec((_DG, _DOUT), lambda i: (0, 0)),
            pl.BlockSpec((_DOUT, 1), lambda i: (0, 0)),
        ],
        out_specs=pl.BlockSpec((_DOUT, _BN), lambda i: (0, i)),
        out_shape=jax.ShapeDtypeStruct((_DOUT, _NPAD), jnp.float32),
    )(a, pt_pad, w, fb)


def kernel(features, positions, neighborhoods, position_theta, position_bias,
           feature_bias):
    f = features[0]
    pos = positions[0]
    nbr = neighborhoods[0]
    pad = _NPAD - _N

    f_pad = jnp.pad(f, ((0, 0), (0, pad)))
    pt_pad = jnp.pad(pos, ((0, 8 - _DP), (0, pad))).T
    w4 = jnp.pad(pos, ((0, 0), (0, pad))).reshape(-1)
    nbr_pm = jnp.pad(nbr, ((0, 0), (0, pad))).T.reshape(-1)

    ft = _build_rows(f_pad)
    a = _sc_gather(ft, w4, nbr_pm)

    theta = position_theta[0]
    w = jnp.concatenate([position_bias[None], theta], axis=0).reshape(_DG, _DOUT)
    y = _contract(a, pt_pad, w, feature_bias)
    return y[None, :, :_N]


# speedup vs baseline: 2.0456x; 2.0456x over previous
import functools

import jax
import jax.numpy as jnp
from jax import lax
from jax.experimental import pallas as pl
from jax.experimental.pallas import tpu as pltpu
from jax.experimental.pallas import tpu_sc as plsc

_N = 10000
_DIN = 128
_DP = 3
_K = 16
_DOUT = 128
_NPAD = 10240
_PT = _NPAD // 32
_PB = 8
_SLOW_CORE = 0
_NSB_S = 40
_NSB_F = 40
_NG = _DP + 1
_DG = _NG * _DIN
_NVR = _DIN // 16
_BN = 1024


def _rows_body(f_ref, ft_ref):
    ft_ref[...] = f_ref[...].T


def _build_rows(f_pad):
    return pl.pallas_call(
        _rows_body,
        grid=(_NPAD // _BN,),
        in_specs=[pl.BlockSpec((_DIN, _BN), lambda i: (0, i))],
        out_specs=pl.BlockSpec((_BN, _DIN), lambda i: (i, 0)),
        out_shape=jax.ShapeDtypeStruct((_NPAD, _DIN), jnp.float32),
    )(f_pad)


def _sc_body(ft_hbm, w4_hbm, nbr_hbm, out_hbm, idx_v, fbufs, w4_v, obufs,
             gsems, osems):
    cid = lax.axis_index("c")
    sid = lax.axis_index("s")
    rows = _PB * _K
    is_slow = cid == _SLOW_CORE
    nsb = jnp.where(is_slow, _NSB_S, _NSB_F)
    base = jnp.where(is_slow, sid * (_NSB_S * _PB),
                     _NSB_S * _PB * 16 + sid * (_NSB_F * _PB))

    with jax.named_scope("sc_init"):
        pltpu.sync_copy(w4_hbm, w4_v)
        pltpu.sync_copy(nbr_hbm.at[pl.ds(base * _K, _NSB_F * _PB * _K)],
                        idx_v)

    def start_gather(s, b):
        pltpu.async_copy(ft_hbm.at[idx_v.at[pl.ds(s * rows, rows)]],
                         fbufs.at[b], gsems[b])

    def compute(s, b):
        with jax.named_scope("sc_compute"):
            _compute_inner(s, b)

    def _compute_inner(s, b):
        @pl.loop(0, _PB)
        def _point(p):
            r0 = p * _K
            idxp = idx_v[pl.ds(s * rows + r0, _K)]
            wvec = [plsc.load_gather(w4_v, [idxp + g * _NPAD])
                    for g in range(_DP)]
            acc = [[jnp.zeros((16,), jnp.float32) for _ in range(_NVR)]
                   for _ in range(_NG)]
            for k in range(_K):
                fv = [fbufs[b, r0 + k, pl.ds(j * 16, 16)]
                      for j in range(_NVR)]
                for j in range(_NVR):
                    acc[0][j] = acc[0][j] + fv[j]
                for g in range(_DP):
                    wgk = wvec[g][k]
                    for j in range(_NVR):
                        acc[g + 1][j] = acc[g + 1][j] + wgk * fv[j]
            for g in range(_NG):
                for j in range(_NVR):
                    obufs[b, p, pl.ds(g * _DIN + j * 16, 16)] = acc[g][j]

    def wait_gather(b):
        with jax.named_scope("sc_wait_gather"):
            pltpu.make_async_copy(ft_hbm.at[idx_v.at[pl.ds(0, rows)]],
                                  fbufs.at[b], gsems[b]).wait()

    def start_store(s, b):
        pltpu.async_copy(obufs.at[b], out_hbm.at[pl.ds(base + s * _PB, _PB)],
                         osems[b])

    def wait_store(s, b):
        pltpu.make_async_copy(obufs.at[b],
                              out_hbm.at[pl.ds(base + s * _PB, _PB)],
                              osems[b]).wait()

    nbuf = 4
    with jax.named_scope("sc_prime"):
        for b in range(nbuf - 1):
            start_gather(b, b)

    @pl.loop(0, nsb, step=nbuf)
    def _quad(s):
        for b in range(nbuf):
            pf = s + b + nbuf - 1

            @pl.when(pf < nsb)
            def _():
                start_gather(pf, (b + nbuf - 1) % nbuf)

            wait_gather(b)

            @pl.when(s >= nbuf)
            def _():
                wait_store(s + b - nbuf, b)

            compute(s + b, b)
            start_store(s + b, b)

    for b in range(nbuf):
        wait_store(nsb - nbuf + b, b)


def _sc_gather(ft, w4, nbr_pm):
    mesh = plsc.VectorSubcoreMesh(core_axis_name="c", subcore_axis_name="s")
    return pl.kernel(
        _sc_body,
        out_type=jax.ShapeDtypeStruct((_NPAD, _DG), jnp.float32),
        mesh=mesh,
        compiler_params=pltpu.CompilerParams(needs_layout_passes=False),
        scratch_types=[
            pltpu.VMEM((_NSB_F * _PB * _K,), jnp.int32),
            pltpu.VMEM((4, _PB * _K, _DIN), jnp.float32),
            pltpu.VMEM((_DP * _NPAD,), jnp.float32),
            pltpu.VMEM((4, _PB, _DG), jnp.float32),
            [pltpu.SemaphoreType.DMA] * 4,
            [pltpu.SemaphoreType.DMA] * 4,
        ],
    )(ft, w4, nbr_pm)


def _y_body(a_ref, pt_ref, w_ref, fb_ref, y_ref):
    a = a_ref[...]
    pt = pt_ref[...]
    a0 = a[:, 0:_DIN]
    parts = [a0]
    for p in range(_DP):
        parts.append(a[:, (p + 1) * _DIN:(p + 2) * _DIN] - a0 * pt[:, p:p + 1])
    ahat = jnp.concatenate(parts, axis=1)
    w = w_ref[...]
    y_t = lax.dot_general(w, ahat, (((0,), (1,)), ((), ())),
                          preferred_element_type=jnp.float32)
    y_ref[...] = y_t + fb_ref[...]


def _contract(a, pt_pad, w, fb):
    return pl.pallas_call(
        _y_body,
        grid=(_NPAD // _BN,),
        in_specs=[
            pl.BlockSpec((_BN, _DG), lambda i: (i, 0)),
            pl.BlockSpec((_BN, 8), lambda i: (i, 0)),
            pl.BlockSpec((_DG, _DOUT), lambda i: (0, 0)),
            pl.BlockSpec((_DOUT, 1), lambda i: (0, 0)),
        ],
        out_specs=pl.BlockSpec((_DOUT, _BN), lambda i: (0, i)),
        out_shape=jax.ShapeDtypeStruct((_DOUT, _NPAD), jnp.float32),
    )(a, pt_pad, w, fb)


def kernel(features, positions, neighborhoods, position_theta, position_bias,
           feature_bias):
    f = features[0]
    pos = positions[0]
    nbr = neighborhoods[0]
    pad = _NPAD - _N

    f_pad = jnp.pad(f, ((0, 0), (0, pad)))
    pt_pad = jnp.pad(pos, ((0, 8 - _DP), (0, pad))).T
    w4 = jnp.pad(pos, ((0, 0), (0, pad))).reshape(-1)
    pad_idx = jnp.broadcast_to(jnp.arange(_N, _NPAD, dtype=jnp.int32),
                               (_K, pad))
    nbr_pm = jnp.concatenate([nbr, pad_idx], axis=1).T.reshape(-1)

    ft = _build_rows(f_pad)
    a = _sc_gather(ft, w4, nbr_pm)

    theta = position_theta[0]
    w = jnp.concatenate([position_bias[None], theta], axis=0).reshape(_DG, _DOUT)
    y = _contract(a, pt_pad, w, feature_bias)
    return y[None, :, :_N]
